# Initial kernel scaffold; baseline (speedup 1.0000x reference)
import functools

import jax
import jax.numpy as jnp
from jax import lax
from jax.experimental import pallas as pl
from jax.experimental.pallas import tpu as pltpu
from jax.experimental.pallas import tpu_sc as plsc

N = 10000
D = 128
E = 320000

NC, NS = 2, 16
NW = NC * NS
B = 128

ETOT = E + N
TILE_E = -(-ETOT // (NW * B)) * B
EPAD = TILE_E * NW
NB = TILE_E // B

NPAD = 10240
NSLICE = NPAD // NS
E16 = EPAD // NS
NB16 = E16 // B

ROWS_T = N // NS

_mesh = plsc.VectorSubcoreMesh(
    core_axis_name="c", subcore_axis_name="s", num_cores=NC, num_subcores=NS)


@functools.partial(
    pl.kernel,
    out_type=jax.ShapeDtypeStruct((EPAD,), jnp.float32),
    mesh=_mesh,
    scratch_types=[
        pltpu.VMEM_SHARED((NPAD,), jnp.float32),
        pltpu.VMEM((B,), jnp.int32),
        pltpu.VMEM((B,), jnp.int32),
        pltpu.VMEM((B,), jnp.float32),
        pltpu.VMEM((B,), jnp.float32),
        pltpu.VMEM((NSLICE,), jnp.float32),
        pltpu.VMEM((NPAD,), jnp.float32),
    ],
)
def _norm_kernel(rows_h, cols_h, ew_h, norm_h, deg_sh, ridx, cidx, ewv, nv,
                 dv, disf):
    c = lax.axis_index("c")
    s = lax.axis_index("s")
    z16 = jnp.zeros((16,), jnp.float32)

    def zb(i, _):
        dv[pl.ds(pl.multiple_of(i * 16, 16), 16)] = z16
        return 0
    lax.fori_loop(0, NSLICE // 16, zb, 0)
    pltpu.sync_copy(dv, deg_sh.at[pl.ds(s * NSLICE, NSLICE)])
    plsc.subcore_barrier()

    def dblk(i, _):
        base = s * E16 + i * B
        pltpu.sync_copy(cols_h.at[pl.ds(base, B)], cidx)
        pltpu.sync_copy(ew_h.at[pl.ds(base, B)], ewv)
        pltpu.sync_copy(ewv, deg_sh.at[cidx], add=True)
        return 0
    lax.fori_loop(0, NB16, dblk, 0)
    plsc.subcore_barrier()

    pltpu.sync_copy(deg_sh.at[pl.ds(s * NSLICE, NSLICE)], dv)
    half = jnp.full((16,), 0.5, jnp.float32)
    th = jnp.full((16,), 1.5, jnp.float32)
    magic = jnp.full((16,), 0x5F3759DF, jnp.int32)
    one16 = jnp.full((16,), 1, jnp.int32)

    def rs(i, _):
        sl = pl.ds(pl.multiple_of(i * 16, 16), 16)
        d = dv[sl]
        iv = magic - lax.shift_right_logical(plsc.bitcast(d, jnp.int32), one16)
        y = plsc.bitcast(iv, jnp.float32)
        hd = half * d
        y = y * (th - hd * y * y)
        y = y * (th - hd * y * y)
        y = y * (th - hd * y * y)
        dv[sl] = y
        return 0
    lax.fori_loop(0, NSLICE // 16, rs, 0)
    pltpu.sync_copy(dv, deg_sh.at[pl.ds(s * NSLICE, NSLICE)])
    plsc.subcore_barrier()

    pltpu.sync_copy(deg_sh, disf)
    w = s * NC + c

    def nblk(i, _):
        base = w * TILE_E + i * B
        pltpu.sync_copy(rows_h.at[pl.ds(base, B)], ridx)
        pltpu.sync_copy(cols_h.at[pl.ds(base, B)], cidx)
        pltpu.sync_copy(ew_h.at[pl.ds(base, B)], ewv)
        for k in range(B // 16):
            sl = pl.ds(k * 16, 16)
            a = plsc.load_gather(disf, [ridx[sl]])
            b = plsc.load_gather(disf, [cidx[sl]])
            nv[sl] = a * ewv[sl] * b
        pltpu.sync_copy(nv, norm_h.at[pl.ds(base, B)])
        return 0
    lax.fori_loop(0, NB, nblk, 0)


@functools.partial(
    pl.kernel,
    out_type=jax.ShapeDtypeStruct((NC, N, D), jnp.float32),
    mesh=_mesh,
    scratch_types=[
        pltpu.VMEM_SHARED((N, D), jnp.float32),
        pltpu.VMEM((B,), jnp.int32),
        pltpu.VMEM((B,), jnp.int32),
        pltpu.VMEM((B,), jnp.float32),
        pltpu.VMEM((B, D), jnp.float32),
        pltpu.SemaphoreType.DMA,
    ],
)
def _mp_kernel(h_h, rows_h, cols_h, norm_h, out_h, acc_sh, ridx, cidx, nv,
               rbuf, sem):
    c = lax.axis_index("c")
    s = lax.axis_index("s")
    z16 = jnp.zeros((16,), jnp.float32)

    def zb(i, _):
        r = i // (D // 16)
        k = i % (D // 16)
        rbuf[r, pl.ds(pl.multiple_of(k * 16, 16), 16)] = z16
        return 0
    lax.fori_loop(0, B * D // 16, zb, 0)
    for j in range(5):
        pltpu.sync_copy(rbuf.at[pl.ds(0, ROWS_T // 5)],
                        acc_sh.at[pl.ds(s * ROWS_T + j * (ROWS_T // 5),
                                        ROWS_T // 5)])
    plsc.subcore_barrier()

    w = s * NC + c

    def blk(i, _):
        base = w * TILE_E + i * B
        pltpu.sync_copy(rows_h.at[pl.ds(base, B)], ridx)
        pltpu.sync_copy(cols_h.at[pl.ds(base, B)], cidx)
        pltpu.sync_copy(norm_h.at[pl.ds(base, B)], nv)
        pltpu.async_copy(h_h.at[ridx], rbuf, sem).wait()

        def ebody(e, _):
            ns = plsc.load_gather(nv, [jnp.full((16,), e, jnp.int32)])
            for k in range(D // 16):
                sl = pl.ds(k * 16, 16)
                rbuf[e, sl] = rbuf[e, sl] * ns
            return 0
        lax.fori_loop(0, B, ebody, 0)
        pltpu.sync_copy(rbuf, acc_sh.at[cidx], add=True)
        return 0
    lax.fori_loop(0, NB, blk, 0)
    plsc.subcore_barrier()

    r0 = s * ROWS_T
    pltpu.sync_copy(acc_sh.at[pl.ds(r0, ROWS_T)],
                    out_h.at[c, pl.ds(r0, ROWS_T)])


def _mm_body(x_ref, w_ref, o_ref):
    o_ref[...] = lax.dot_general(
        x_ref[...], w_ref[...], (((1,), (1,)), ((), ())),
        preferred_element_type=jnp.float32)


def _tc_matmul(x, W):
    return pl.pallas_call(
        _mm_body,
        grid=(10,),
        in_specs=[pl.BlockSpec((N // 10, D), lambda i: (i, 0)),
                  pl.BlockSpec((D, D), lambda i: (0, 0))],
        out_specs=pl.BlockSpec((N // 10, D), lambda i: (i, 0)),
        out_shape=jax.ShapeDtypeStruct((N, D), jnp.float32),
    )(x, W)


def _mm2_body(p_ref, b_ref, w_ref, o_ref):
    t = jnp.maximum(p_ref[0] + p_ref[1] + b_ref[...], 0.0)
    o_ref[...] = lax.dot_general(
        t, w_ref[...], (((1,), (1,)), ((), ())),
        preferred_element_type=jnp.float32)


def _tc_combine_matmul(p, b, W):
    return pl.pallas_call(
        _mm2_body,
        grid=(10,),
        in_specs=[pl.BlockSpec((NC, N // 10, D), lambda i: (0, i, 0)),
                  pl.BlockSpec((1, D), lambda i: (0, 0)),
                  pl.BlockSpec((D, D), lambda i: (0, 0))],
        out_specs=pl.BlockSpec((N // 10, D), lambda i: (i, 0)),
        out_shape=jax.ShapeDtypeStruct((N, D), jnp.float32),
    )(p, b, W)


def _fin_body(p_ref, b_ref, o_ref):
    o_ref[...] = jnp.maximum(p_ref[0] + p_ref[1] + b_ref[...], 0.0)


def _tc_combine_relu(p, b):
    return pl.pallas_call(
        _fin_body,
        grid=(10,),
        in_specs=[pl.BlockSpec((NC, N // 10, D), lambda i: (0, i, 0)),
                  pl.BlockSpec((1, D), lambda i: (0, 0))],
        out_specs=pl.BlockSpec((N // 10, D), lambda i: (i, 0)),
        out_shape=jax.ShapeDtypeStruct((N, D), jnp.float32),
    )(p, b)


def kernel(x, edge_index, edge_weights, W1, b1, W2, b2):
    row = edge_index[0]
    col = edge_index[1]
    loop = jnp.arange(N, dtype=row.dtype)
    npad = EPAD - ETOT
    rows_all = jnp.concatenate([row, loop, jnp.zeros((npad,), row.dtype)])
    cols_all = jnp.concatenate([col, loop, jnp.zeros((npad,), col.dtype)])
    ew_all = jnp.concatenate([edge_weights, jnp.ones((N,), jnp.float32),
                              jnp.zeros((npad,), jnp.float32)])

    norm = _norm_kernel(rows_all, cols_all, ew_all)

    h1 = _tc_matmul(x, W1)
    p1 = _mp_kernel(h1, rows_all, cols_all, norm)
    h2 = _tc_combine_matmul(p1, b1.reshape(1, D), W2)
    p2 = _mp_kernel(h2, rows_all, cols_all, norm)
    return _tc_combine_relu(p2, b2.reshape(1, D))


# speedup vs baseline: 7.9574x; 7.9574x over previous
import functools

import jax
import jax.numpy as jnp
from jax import lax
from jax.experimental import pallas as pl
from jax.experimental.pallas import tpu as pltpu
from jax.experimental.pallas import tpu_sc as plsc

N = 10000
D = 128
E = 320000

NC, NS = 2, 16
NW = NC * NS
B = 128

ETOT = E + N
TILE_E = -(-ETOT // (NW * B)) * B
EPAD = TILE_E * NW
NB = TILE_E // B

NPAD = 10240
NSLICE = NPAD // NS
E16 = EPAD // NS
NB16 = E16 // B

ROWS_T = N // NS

_mesh = plsc.VectorSubcoreMesh(
    core_axis_name="c", subcore_axis_name="s", num_cores=NC, num_subcores=NS)


@functools.partial(
    pl.kernel,
    out_type=jax.ShapeDtypeStruct((EPAD,), jnp.float32),
    mesh=_mesh,
    compiler_params=pltpu.CompilerParams(needs_layout_passes=False),
    scratch_types=[
        pltpu.VMEM_SHARED((NPAD,), jnp.float32),
        pltpu.VMEM((B,), jnp.int32),
        pltpu.VMEM((B,), jnp.int32),
        pltpu.VMEM((B,), jnp.float32),
        pltpu.VMEM((B,), jnp.float32),
        pltpu.VMEM((NSLICE,), jnp.float32),
        pltpu.VMEM((NPAD,), jnp.float32),
    ],
)
def _norm_kernel(rows_h, cols_h, ew_h, norm_h, deg_sh, ridx, cidx, ewv, nv,
                 dv, disf):
    c = lax.axis_index("c")
    s = lax.axis_index("s")
    z16 = jnp.zeros((16,), jnp.float32)

    def zb(i, _):
        dv[pl.ds(pl.multiple_of(i * 16, 16), 16)] = z16
        return 0
    lax.fori_loop(0, NSLICE // 16, zb, 0)
    pltpu.sync_copy(dv, deg_sh.at[pl.ds(s * NSLICE, NSLICE)])
    plsc.subcore_barrier()

    def dblk(i, _):
        base = s * E16 + i * B
        pltpu.sync_copy(cols_h.at[pl.ds(base, B)], cidx)
        pltpu.sync_copy(ew_h.at[pl.ds(base, B)], ewv)
        pltpu.sync_copy(ewv, deg_sh.at[cidx], add=True)
        return 0
    lax.fori_loop(0, NB16, dblk, 0)
    plsc.subcore_barrier()

    pltpu.sync_copy(deg_sh.at[pl.ds(s * NSLICE, NSLICE)], dv)
    half = jnp.full((16,), 0.5, jnp.float32)
    th = jnp.full((16,), 1.5, jnp.float32)
    one = jnp.full((16,), 1.0, jnp.float32)

    def rs(i, _):
        sl = pl.ds(pl.multiple_of(i * 16, 16), 16)
        d = jnp.maximum(dv[sl], one)
        y = one / d
        hd = half * d
        for _ in range(24):
            y = y * (th - hd * y * y)
        dv[sl] = y
        return 0
    lax.fori_loop(0, NSLICE // 16, rs, 0)
    pltpu.sync_copy(dv, deg_sh.at[pl.ds(s * NSLICE, NSLICE)])
    plsc.subcore_barrier()

    pltpu.sync_copy(deg_sh, disf)
    w = s * NC + c

    def nblk(i, _):
        base = w * TILE_E + i * B
        pltpu.sync_copy(rows_h.at[pl.ds(base, B)], ridx)
        pltpu.sync_copy(cols_h.at[pl.ds(base, B)], cidx)
        pltpu.sync_copy(ew_h.at[pl.ds(base, B)], ewv)
        for k in range(B // 16):
            sl = pl.ds(k * 16, 16)
            a = plsc.load_gather(disf, [ridx[sl]])
            b = plsc.load_gather(disf, [cidx[sl]])
            nv[sl] = a * ewv[sl] * b
        pltpu.sync_copy(nv, norm_h.at[pl.ds(base, B)])
        return 0
    lax.fori_loop(0, NB, nblk, 0)


@functools.partial(
    pl.kernel,
    out_type=jax.ShapeDtypeStruct((NC, N, D), jnp.float32),
    mesh=_mesh,
    compiler_params=pltpu.CompilerParams(needs_layout_passes=False),
    scratch_types=[
        pltpu.VMEM_SHARED((N, D), jnp.float32),
        pltpu.VMEM((B,), jnp.int32),
        pltpu.VMEM((B,), jnp.int32),
        pltpu.VMEM((B,), jnp.float32),
        pltpu.VMEM((B, D), jnp.float32),
        pltpu.SemaphoreType.DMA,
    ],
)
def _mp_kernel(h_h, rows_h, cols_h, norm_h, out_h, acc_sh, ridx, cidx, nv,
               rbuf, sem):
    c = lax.axis_index("c")
    s = lax.axis_index("s")
    z16 = jnp.zeros((16,), jnp.float32)

    def zb(i, _):
        r = i // (D // 16)
        k = i % (D // 16)
        rbuf[r, pl.ds(pl.multiple_of(k * 16, 16), 16)] = z16
        return 0
    lax.fori_loop(0, B * D // 16, zb, 0)

    @pl.when(s < NS - 1)
    def _():
        for j in range(4):
            pltpu.sync_copy(rbuf, acc_sh.at[pl.ds(s * 624 + j * B, B)])
        pltpu.sync_copy(rbuf.at[pl.ds(0, 112)],
                        acc_sh.at[pl.ds(s * 624 + 4 * B, 112)])

    @pl.when(s == NS - 1)
    def _():
        for j in range(5):
            pltpu.sync_copy(rbuf, acc_sh.at[pl.ds(15 * 624 + j * B, B)])

    plsc.subcore_barrier()

    w = s * NC + c

    def blk(i, _):
        base = w * TILE_E + i * B
        pltpu.sync_copy(rows_h.at[pl.ds(base, B)], ridx)
        pltpu.sync_copy(cols_h.at[pl.ds(base, B)], cidx)
        pltpu.sync_copy(norm_h.at[pl.ds(base, B)], nv)
        pltpu.async_copy(h_h.at[ridx], rbuf, sem).wait()

        def ebody(e, _):
            ns = plsc.load_gather(nv, [jnp.full((16,), e, jnp.int32)])
            for k in range(D // 16):
                sl = pl.ds(k * 16, 16)
                rbuf[e, sl] = rbuf[e, sl] * ns
            return 0
        lax.fori_loop(0, B, ebody, 0)
        pltpu.sync_copy(rbuf, acc_sh.at[cidx], add=True)
        return 0
    lax.fori_loop(0, NB, blk, 0)
    plsc.subcore_barrier()

    @pl.when(s < NS - 1)
    def _():
        pltpu.sync_copy(acc_sh.at[pl.ds(s * 624, 624)],
                        out_h.at[c, pl.ds(s * 624, 624)])

    @pl.when(s == NS - 1)
    def _():
        pltpu.sync_copy(acc_sh.at[pl.ds(15 * 624, 640)],
                        out_h.at[c, pl.ds(15 * 624, 640)])


def _mm_body(x_ref, w_ref, o_ref):
    o_ref[...] = lax.dot_general(
        x_ref[...], w_ref[...], (((1,), (1,)), ((), ())),
        preferred_element_type=jnp.float32)


def _tc_matmul(x, W):
    return pl.pallas_call(
        _mm_body,
        grid=(10,),
        in_specs=[pl.BlockSpec((N // 10, D), lambda i: (i, 0)),
                  pl.BlockSpec((D, D), lambda i: (0, 0))],
        out_specs=pl.BlockSpec((N // 10, D), lambda i: (i, 0)),
        out_shape=jax.ShapeDtypeStruct((N, D), jnp.float32),
    )(x, W)


def _mm2_body(p_ref, b_ref, w_ref, o_ref):
    t = jnp.maximum(p_ref[0] + p_ref[1] + b_ref[...], 0.0)
    o_ref[...] = lax.dot_general(
        t, w_ref[...], (((1,), (1,)), ((), ())),
        preferred_element_type=jnp.float32)


def _tc_combine_matmul(p, b, W):
    return pl.pallas_call(
        _mm2_body,
        grid=(10,),
        in_specs=[pl.BlockSpec((NC, N // 10, D), lambda i: (0, i, 0)),
                  pl.BlockSpec((1, D), lambda i: (0, 0)),
                  pl.BlockSpec((D, D), lambda i: (0, 0))],
        out_specs=pl.BlockSpec((N // 10, D), lambda i: (i, 0)),
        out_shape=jax.ShapeDtypeStruct((N, D), jnp.float32),
    )(p, b, W)


def _fin_body(p_ref, b_ref, o_ref):
    o_ref[...] = jnp.maximum(p_ref[0] + p_ref[1] + b_ref[...], 0.0)


def _tc_combine_relu(p, b):
    return pl.pallas_call(
        _fin_body,
        grid=(10,),
        in_specs=[pl.BlockSpec((NC, N // 10, D), lambda i: (0, i, 0)),
                  pl.BlockSpec((1, D), lambda i: (0, 0))],
        out_specs=pl.BlockSpec((N // 10, D), lambda i: (i, 0)),
        out_shape=jax.ShapeDtypeStruct((N, D), jnp.float32),
    )(p, b)


def kernel(x, edge_index, edge_weights, W1, b1, W2, b2):
    row = edge_index[0]
    col = edge_index[1]
    loop = jnp.arange(N, dtype=row.dtype)
    npad = EPAD - ETOT
    rows_all = jnp.concatenate([row, loop, jnp.zeros((npad,), row.dtype)])
    cols_all = jnp.concatenate([col, loop, jnp.zeros((npad,), col.dtype)])
    ew_all = jnp.concatenate([edge_weights, jnp.ones((N,), jnp.float32),
                              jnp.zeros((npad,), jnp.float32)])

    norm = _norm_kernel(rows_all, cols_all, ew_all)

    h1 = _tc_matmul(x, W1)
    p1 = _mp_kernel(h1, rows_all, cols_all, norm)
    h2 = _tc_combine_matmul(p1, b1.reshape(1, D), W2)
    p2 = _mp_kernel(h2, rows_all, cols_all, norm)
    return _tc_combine_relu(p2, b2.reshape(1, D))


# speedup vs baseline: 22.3145x; 2.8042x over previous
import functools

import jax
import jax.numpy as jnp
from jax import lax
from jax.experimental import pallas as pl
from jax.experimental.pallas import tpu as pltpu
from jax.experimental.pallas import tpu_sc as plsc

N = 10000
D = 128
E = 320000

NC, NS = 2, 16
NW = NC * NS
B = 128

ETOT = E + N
TILE_E = -(-ETOT // (NW * B)) * B
EPAD = TILE_E * NW
NB = TILE_E // B

NPAD = 10240
NSLICE = NPAD // NS
E16 = EPAD // NS
NB16 = E16 // B

_mesh = plsc.VectorSubcoreMesh(
    core_axis_name="c", subcore_axis_name="s", num_cores=NC, num_subcores=NS)


@functools.partial(
    pl.kernel,
    out_type=jax.ShapeDtypeStruct((EPAD,), jnp.float32),
    mesh=_mesh,
    compiler_params=pltpu.CompilerParams(needs_layout_passes=False),
    scratch_types=[
        pltpu.VMEM_SHARED((NS, NPAD), jnp.float32),
        pltpu.VMEM_SHARED((NPAD,), jnp.float32),
        pltpu.VMEM((NPAD,), jnp.float32),
        pltpu.VMEM((2, B), jnp.int32),
        pltpu.VMEM((2, B), jnp.int32),
        pltpu.VMEM((2, B), jnp.float32),
        pltpu.VMEM((2, B), jnp.float32),
        pltpu.VMEM((NSLICE,), jnp.float32),
        pltpu.VMEM((NSLICE,), jnp.float32),
        pltpu.VMEM((NPAD,), jnp.float32),
        pltpu.SemaphoreType.DMA((2,)),
        pltpu.SemaphoreType.DMA((2,)),
    ],
)
def _norm_kernel(rows_h, cols_h, ew_h, norm_h, degs_sh, dis_sh, degacc,
                 ridx, cidx, ewv, nvout, dv, tv, disf, semi, semo):
    c = lax.axis_index("c")
    s = lax.axis_index("s")
    z16 = jnp.zeros((16,), jnp.float32)

    def zb(i, _):
        degacc[pl.ds(pl.multiple_of(i * 16, 16), 16)] = z16
        return 0
    lax.fori_loop(0, NPAD // 16, zb, 0)

    def p1_load(bi, slot):
        base = s * E16 + bi * B
        pltpu.async_copy(cols_h.at[pl.ds(base, B)], cidx.at[slot],
                         semi.at[slot])
        pltpu.async_copy(ew_h.at[pl.ds(base, B)], ewv.at[slot],
                         semi.at[slot])

    def p1_wait(slot):
        pltpu.make_async_copy(cols_h.at[pl.ds(0, B)], cidx.at[slot],
                              semi.at[slot]).wait()
        pltpu.make_async_copy(ew_h.at[pl.ds(0, B)], ewv.at[slot],
                              semi.at[slot]).wait()

    p1_load(0, 0)

    def dblk(i, _):
        sl_i = i % 2

        @pl.when(i + 1 < NB16)
        def _():
            p1_load(i + 1, (i + 1) % 2)
        p1_wait(sl_i)
        for k in range(B // 16):
            ksl = pl.ds(k * 16, 16)
            plsc.addupdate_scatter(degacc, [cidx[sl_i, ksl]], ewv[sl_i, ksl])
        return 0
    lax.fori_loop(0, NB16, dblk, 0)

    pltpu.sync_copy(degacc, degs_sh.at[s])
    plsc.subcore_barrier()

    pltpu.sync_copy(degs_sh.at[0, pl.ds(s * NSLICE, NSLICE)], dv)
    for t in range(1, NS):
        pltpu.sync_copy(degs_sh.at[t, pl.ds(s * NSLICE, NSLICE)], tv)

        def radd(i, _):
            ksl = pl.ds(pl.multiple_of(i * 16, 16), 16)
            dv[ksl] = dv[ksl] + tv[ksl]
            return 0
        lax.fori_loop(0, NSLICE // 16, radd, 0)

    half = jnp.full((16,), 0.5, jnp.float32)
    th = jnp.full((16,), 1.5, jnp.float32)
    one = jnp.full((16,), 1.0, jnp.float32)

    def rs(i, _):
        ksl = pl.ds(pl.multiple_of(i * 16, 16), 16)
        d = jnp.maximum(dv[ksl], one)
        y = one / d
        hd = half * d
        for _ in range(24):
            y = y * (th - hd * y * y)
        dv[ksl] = y
        return 0
    lax.fori_loop(0, NSLICE // 16, rs, 0)
    pltpu.sync_copy(dv, dis_sh.at[pl.ds(s * NSLICE, NSLICE)])
    plsc.subcore_barrier()

    pltpu.sync_copy(dis_sh, disf)
    w = s * NC + c
    g0 = w * TILE_E

    def p3_load(bi, slot):
        base = g0 + bi * B
        pltpu.async_copy(rows_h.at[pl.ds(base, B)], ridx.at[slot],
                         semi.at[slot])
        pltpu.async_copy(cols_h.at[pl.ds(base, B)], cidx.at[slot],
                         semi.at[slot])
        pltpu.async_copy(ew_h.at[pl.ds(base, B)], ewv.at[slot],
                         semi.at[slot])

    def p3_wait(slot):
        pltpu.make_async_copy(rows_h.at[pl.ds(0, B)], ridx.at[slot],
                              semi.at[slot]).wait()
        pltpu.make_async_copy(cols_h.at[pl.ds(0, B)], cidx.at[slot],
                              semi.at[slot]).wait()
        pltpu.make_async_copy(ew_h.at[pl.ds(0, B)], ewv.at[slot],
                              semi.at[slot]).wait()

    p3_load(0, 0)

    def nblk(i, _):
        sl_i = i % 2

        @pl.when(i + 1 < NB)
        def _():
            p3_load(i + 1, (i + 1) % 2)
        p3_wait(sl_i)

        @pl.when(i >= 2)
        def _():
            pltpu.make_async_copy(nvout.at[sl_i],
                                  norm_h.at[pl.ds(0, B)],
                                  semo.at[sl_i]).wait()
        for k in range(B // 16):
            ksl = pl.ds(k * 16, 16)
            a = plsc.load_gather(disf, [ridx[sl_i, ksl]])
            b = plsc.load_gather(disf, [cidx[sl_i, ksl]])
            nvout[sl_i, ksl] = a * ewv[sl_i, ksl] * b
        pltpu.async_copy(nvout.at[sl_i], norm_h.at[pl.ds(g0 + i * B, B)],
                         semo.at[sl_i])
        return 0
    lax.fori_loop(0, NB, nblk, 0)
    for t in range(2):
        pltpu.make_async_copy(nvout.at[t], norm_h.at[pl.ds(0, B)],
                              semo.at[t]).wait()


@functools.partial(
    pl.kernel,
    out_type=jax.ShapeDtypeStruct((NC, N, D), jnp.float32),
    mesh=_mesh,
    compiler_params=pltpu.CompilerParams(needs_layout_passes=False),
    scratch_types=[
        pltpu.VMEM_SHARED((N, D), jnp.float32),
        pltpu.VMEM((3, B), jnp.int32),
        pltpu.VMEM((4, B), jnp.int32),
        pltpu.VMEM((3, B), jnp.float32),
        pltpu.VMEM((3, B, D), jnp.float32),
        pltpu.SemaphoreType.DMA((3,)),
        pltpu.SemaphoreType.DMA,
        pltpu.SemaphoreType.DMA((2,)),
    ],
)
def _mp_kernel(h_h, rows_h, cols_h, norm_h, out_h, acc_sh, ridx, cidx, nv,
               rbuf, semi, semg, sems):
    c = lax.axis_index("c")
    s = lax.axis_index("s")
    z16 = jnp.zeros((16,), jnp.float32)

    def zb(i, _):
        r = i // (D // 16)
        k = i % (D // 16)
        rbuf[0, r, pl.ds(pl.multiple_of(k * 16, 16), 16)] = z16
        return 0
    lax.fori_loop(0, B * D // 16, zb, 0)

    @pl.when(s < NS - 1)
    def _():
        for j in range(4):
            pltpu.sync_copy(rbuf.at[0], acc_sh.at[pl.ds(s * 624 + j * B, B)])
        pltpu.sync_copy(rbuf.at[0, pl.ds(0, 112)],
                        acc_sh.at[pl.ds(s * 624 + 4 * B, 112)])

    @pl.when(s == NS - 1)
    def _():
        for j in range(5):
            pltpu.sync_copy(rbuf.at[0], acc_sh.at[pl.ds(15 * 624 + j * B, B)])

    plsc.subcore_barrier()

    w = s * NC + c
    g0 = w * TILE_E

    def idx_load(bi, gen):
        base = g0 + bi * B
        pltpu.async_copy(rows_h.at[pl.ds(base, B)], ridx.at[gen % 3],
                         semi.at[gen % 3])
        pltpu.async_copy(cols_h.at[pl.ds(base, B)], cidx.at[gen % 4],
                         semi.at[gen % 3])
        pltpu.async_copy(norm_h.at[pl.ds(base, B)], nv.at[gen % 3],
                         semi.at[gen % 3])

    def idx_wait(gen):
        pltpu.make_async_copy(rows_h.at[pl.ds(0, B)], ridx.at[gen % 3],
                              semi.at[gen % 3]).wait()
        pltpu.make_async_copy(cols_h.at[pl.ds(0, B)], cidx.at[gen % 4],
                              semi.at[gen % 3]).wait()
        pltpu.make_async_copy(norm_h.at[pl.ds(0, B)], nv.at[gen % 3],
                              semi.at[gen % 3]).wait()

    pltpu.sync_copy(rows_h.at[pl.ds(g0, B)], ridx.at[0])
    pltpu.sync_copy(cols_h.at[pl.ds(g0, B)], cidx.at[0])
    pltpu.sync_copy(norm_h.at[pl.ds(g0, B)], nv.at[0])
    idx_load(1, 1)
    pltpu.async_copy(h_h.at[ridx.at[0]], rbuf.at[0], semg)

    def blk(i, _):
        j3 = i % 3

        @pl.when(i >= 2)
        def _():
            pltpu.make_async_copy(
                rbuf.at[(i - 2) % 3],
                acc_sh.at[cidx.at[(i - 2) % 4]],
                sems.at[i % 2]).wait()

        pltpu.make_async_copy(h_h.at[ridx.at[j3]], rbuf.at[j3], semg).wait()

        @pl.when(i + 1 < NB)
        def _():
            idx_wait(i + 1)
            pltpu.async_copy(h_h.at[ridx.at[(i + 1) % 3]],
                             rbuf.at[(i + 1) % 3], semg)

        @pl.when(i + 2 < NB)
        def _():
            idx_load(i + 2, i + 2)

        @plsc.parallel_loop(0, B, step=1, unroll=4)
        def scale(e):
            ns = plsc.load_gather(nv.at[j3], [jnp.full((16,), e, jnp.int32)])
            for kk in range(D // 16):
                ksl = pl.ds(kk * 16, 16)
                rbuf[j3, e, ksl] = rbuf[j3, e, ksl] * ns

        pltpu.async_copy(rbuf.at[j3], acc_sh.at[cidx.at[i % 4]],
                         sems.at[i % 2], add=True)
        return 0
    lax.fori_loop(0, NB, blk, 0)

    for t in range(2):
        i = NB - 2 + t
        pltpu.make_async_copy(rbuf.at[i % 3], acc_sh.at[cidx.at[i % 4]],
                              sems.at[i % 2]).wait()
    plsc.subcore_barrier()

    @pl.when(s < NS - 1)
    def _():
        pltpu.sync_copy(acc_sh.at[pl.ds(s * 624, 624)],
                        out_h.at[c, pl.ds(s * 624, 624)])

    @pl.when(s == NS - 1)
    def _():
        pltpu.sync_copy(acc_sh.at[pl.ds(15 * 624, 640)],
                        out_h.at[c, pl.ds(15 * 624, 640)])


def _mm_body(x_ref, w_ref, o_ref):
    o_ref[...] = lax.dot_general(
        x_ref[...], w_ref[...], (((1,), (1,)), ((), ())),
        preferred_element_type=jnp.float32)


def _tc_matmul(x, W):
    return pl.pallas_call(
        _mm_body,
        grid=(10,),
        in_specs=[pl.BlockSpec((N // 10, D), lambda i: (i, 0)),
                  pl.BlockSpec((D, D), lambda i: (0, 0))],
        out_specs=pl.BlockSpec((N // 10, D), lambda i: (i, 0)),
        out_shape=jax.ShapeDtypeStruct((N, D), jnp.float32),
    )(x, W)


def _mm2_body(p_ref, b_ref, w_ref, o_ref):
    t = jnp.maximum(p_ref[0] + p_ref[1] + b_ref[...], 0.0)
    o_ref[...] = lax.dot_general(
        t, w_ref[...], (((1,), (1,)), ((), ())),
        preferred_element_type=jnp.float32)


def _tc_combine_matmul(p, b, W):
    return pl.pallas_call(
        _mm2_body,
        grid=(10,),
        in_specs=[pl.BlockSpec((NC, N // 10, D), lambda i: (0, i, 0)),
                  pl.BlockSpec((1, D), lambda i: (0, 0)),
                  pl.BlockSpec((D, D), lambda i: (0, 0))],
        out_specs=pl.BlockSpec((N // 10, D), lambda i: (i, 0)),
        out_shape=jax.ShapeDtypeStruct((N, D), jnp.float32),
    )(p, b, W)


def _fin_body(p_ref, b_ref, o_ref):
    o_ref[...] = jnp.maximum(p_ref[0] + p_ref[1] + b_ref[...], 0.0)


def _tc_combine_relu(p, b):
    return pl.pallas_call(
        _fin_body,
        grid=(10,),
        in_specs=[pl.BlockSpec((NC, N // 10, D), lambda i: (0, i, 0)),
                  pl.BlockSpec((1, D), lambda i: (0, 0))],
        out_specs=pl.BlockSpec((N // 10, D), lambda i: (i, 0)),
        out_shape=jax.ShapeDtypeStruct((N, D), jnp.float32),
    )(p, b)


def kernel(x, edge_index, edge_weights, W1, b1, W2, b2):
    row = edge_index[0]
    col = edge_index[1]
    loop = jnp.arange(N, dtype=row.dtype)
    npad = EPAD - ETOT
    pad_idx = jnp.arange(npad, dtype=row.dtype) % N
    rows_all = jnp.concatenate([row, loop, pad_idx])
    cols_all = jnp.concatenate([col, loop, pad_idx])
    ew_all = jnp.concatenate([edge_weights, jnp.ones((N,), jnp.float32),
                              jnp.zeros((npad,), jnp.float32)])

    norm = _norm_kernel(rows_all, cols_all, ew_all)

    h1 = _tc_matmul(x, W1)
    p1 = _mp_kernel(h1, rows_all, cols_all, norm)
    h2 = _tc_combine_matmul(p1, b1.reshape(1, D), W2)
    p2 = _mp_kernel(h2, rows_all, cols_all, norm)
    return _tc_combine_relu(p2, b2.reshape(1, D))


# speedup vs baseline: 25.4201x; 1.1392x over previous
import functools

import jax
import jax.numpy as jnp
from jax import lax
from jax.experimental import pallas as pl
from jax.experimental.pallas import tpu as pltpu
from jax.experimental.pallas import tpu_sc as plsc

N = 10000
D = 128
E = 320000

NC, NS = 2, 16
NW = NC * NS
B = 128

ETOT = E + N
TILE_E = -(-ETOT // (NW * B)) * B
EPAD = TILE_E * NW
NB = TILE_E // B
TOTB = EPAD // B

NPAD = 10240
NSLICE = NPAD // NS
NB16 = 2 * NB

_mesh = plsc.VectorSubcoreMesh(
    core_axis_name="c", subcore_axis_name="s", num_cores=NC, num_subcores=NS)


@functools.partial(
    pl.kernel,
    out_type=jax.ShapeDtypeStruct((NW, NB, B), jnp.float32),
    mesh=_mesh,
    compiler_params=pltpu.CompilerParams(needs_layout_passes=False),
    scratch_types=[
        pltpu.VMEM_SHARED((NS, NPAD), jnp.float32),
        pltpu.VMEM_SHARED((NPAD,), jnp.float32),
        pltpu.VMEM((NPAD,), jnp.float32),
        pltpu.VMEM((NB16, B), jnp.int32),
        pltpu.VMEM((NB16, B), jnp.float32),
        pltpu.VMEM((NB, B), jnp.int32),
        pltpu.VMEM((NB, B), jnp.float32),
        pltpu.VMEM((NSLICE,), jnp.float32),
        pltpu.VMEM((NSLICE,), jnp.float32),
        pltpu.VMEM((NPAD,), jnp.float32),
    ],
)
def _norm_kernel(rows_h, cols_h, ew_h, norm_h, degs_sh, dis_sh, degacc,
                 c16, w16, r32, nst, dv, tv, disf):
    c = lax.axis_index("c")
    s = lax.axis_index("s")
    z16 = jnp.zeros((16,), jnp.float32)

    w = s * NC + c
    pltpu.sync_copy(cols_h.at[s], c16)
    pltpu.sync_copy(ew_h.at[s], w16)
    pltpu.sync_copy(rows_h.at[w], r32)

    def zb(i, _):
        degacc[pl.ds(pl.multiple_of(i * 16, 16), 16)] = z16
        return 0
    lax.fori_loop(0, NPAD // 16, zb, 0)

    def dblk(i, _):
        for k in range(B // 16):
            ksl = pl.ds(k * 16, 16)
            plsc.addupdate_scatter(degacc, [c16[i, ksl]], w16[i, ksl])
        return 0
    lax.fori_loop(0, NB16, dblk, 0)

    pltpu.sync_copy(degacc, degs_sh.at[s])
    plsc.subcore_barrier()

    pltpu.sync_copy(degs_sh.at[0, pl.ds(s * NSLICE, NSLICE)], dv)
    for t in range(1, NS):
        pltpu.sync_copy(degs_sh.at[t, pl.ds(s * NSLICE, NSLICE)], tv)

        def radd(i, _):
            ksl = pl.ds(pl.multiple_of(i * 16, 16), 16)
            dv[ksl] = dv[ksl] + tv[ksl]
            return 0
        lax.fori_loop(0, NSLICE // 16, radd, 0)

    half = jnp.full((16,), 0.5, jnp.float32)
    th = jnp.full((16,), 1.5, jnp.float32)
    one = jnp.full((16,), 1.0, jnp.float32)

    def rs(i, _):
        ksl = pl.ds(pl.multiple_of(i * 16, 16), 16)
        d = jnp.maximum(dv[ksl], one)
        y = one / d
        hd = half * d
        for _ in range(24):
            y = y * (th - hd * y * y)
        dv[ksl] = y
        return 0
    lax.fori_loop(0, NSLICE // 16, rs, 0)
    pltpu.sync_copy(dv, dis_sh.at[pl.ds(s * NSLICE, NSLICE)])
    plsc.subcore_barrier()

    pltpu.sync_copy(dis_sh, disf)

    def nblk(i, _):
        for k in range(B // 16):
            ksl = pl.ds(k * 16, 16)
            a = plsc.load_gather(disf, [r32[i, ksl]])
            b = plsc.load_gather(disf, [c16[c * NB + i, ksl]])
            nst[i, ksl] = a * w16[c * NB + i, ksl] * b
        return 0
    lax.fori_loop(0, NB, nblk, 0)
    pltpu.sync_copy(nst, norm_h.at[w])


@functools.partial(
    pl.kernel,
    out_type=jax.ShapeDtypeStruct((NC, N, D), jnp.float32),
    mesh=_mesh,
    compiler_params=pltpu.CompilerParams(needs_layout_passes=False),
    scratch_types=[
        pltpu.VMEM_SHARED((N, D), jnp.float32),
        pltpu.VMEM((3, B), jnp.int32),
        pltpu.VMEM((4, B), jnp.int32),
        pltpu.VMEM((3, B), jnp.float32),
        pltpu.VMEM((3, B, D), jnp.float32),
        pltpu.SemaphoreType.DMA((3,)),
        pltpu.SemaphoreType.DMA,
        pltpu.SemaphoreType.DMA((2,)),
    ],
)
def _mp_kernel(h_h, rows_h, cols_h, norm_h, out_h, acc_sh, ridx, cidx, nv,
               rbuf, semi, semg, sems):
    c = lax.axis_index("c")
    s = lax.axis_index("s")
    z16 = jnp.zeros((16,), jnp.float32)

    def zb(i, _):
        r = i // (D // 16)
        k = i % (D // 16)
        rbuf[0, r, pl.ds(pl.multiple_of(k * 16, 16), 16)] = z16
        return 0
    lax.fori_loop(0, B * D // 16, zb, 0)

    @pl.when(s < NS - 1)
    def _():
        for j in range(4):
            pltpu.sync_copy(rbuf.at[0], acc_sh.at[pl.ds(s * 624 + j * B, B)])
        pltpu.sync_copy(rbuf.at[0, pl.ds(0, 112)],
                        acc_sh.at[pl.ds(s * 624 + 4 * B, 112)])

    @pl.when(s == NS - 1)
    def _():
        for j in range(5):
            pltpu.sync_copy(rbuf.at[0], acc_sh.at[pl.ds(15 * 624 + j * B, B)])

    plsc.subcore_barrier()

    g0 = (s * NC + c) * TILE_E

    def idx_load(bi, gen):
        base = g0 + bi * B
        pltpu.async_copy(rows_h.at[pl.ds(base, B)], ridx.at[gen % 3],
                         semi.at[gen % 3])
        pltpu.async_copy(cols_h.at[pl.ds(base, B)], cidx.at[gen % 4],
                         semi.at[gen % 3])
        pltpu.async_copy(norm_h.at[pl.ds(base, B)], nv.at[gen % 3],
                         semi.at[gen % 3])

    def idx_wait(gen):
        pltpu.make_async_copy(rows_h.at[pl.ds(0, B)], ridx.at[gen % 3],
                              semi.at[gen % 3]).wait()
        pltpu.make_async_copy(cols_h.at[pl.ds(0, B)], cidx.at[gen % 4],
                              semi.at[gen % 3]).wait()
        pltpu.make_async_copy(norm_h.at[pl.ds(0, B)], nv.at[gen % 3],
                              semi.at[gen % 3]).wait()

    pltpu.sync_copy(rows_h.at[pl.ds(g0, B)], ridx.at[0])
    pltpu.sync_copy(cols_h.at[pl.ds(g0, B)], cidx.at[0])
    pltpu.sync_copy(norm_h.at[pl.ds(g0, B)], nv.at[0])
    idx_load(1, 1)
    pltpu.async_copy(h_h.at[ridx.at[0]], rbuf.at[0], semg)

    def blk(i, _):
        j3 = i % 3

        @pl.when(i >= 2)
        def _():
            pltpu.make_async_copy(
                rbuf.at[(i - 2) % 3],
                acc_sh.at[cidx.at[(i - 2) % 4]],
                sems.at[i % 2]).wait()

        pltpu.make_async_copy(h_h.at[ridx.at[j3]], rbuf.at[j3], semg).wait()

        @pl.when(i + 1 < NB)
        def _():
            idx_wait(i + 1)
            pltpu.async_copy(h_h.at[ridx.at[(i + 1) % 3]],
                             rbuf.at[(i + 1) % 3], semg)

        @pl.when(i + 2 < NB)
        def _():
            idx_load(i + 2, i + 2)

        @plsc.parallel_loop(0, B, step=1, unroll=8)
        def scale(e):
            ns = plsc.load_gather(nv.at[j3], [jnp.full((16,), e, jnp.int32)])
            for kk in range(D // 16):
                ksl = pl.ds(kk * 16, 16)
                rbuf[j3, e, ksl] = rbuf[j3, e, ksl] * ns

        pltpu.async_copy(rbuf.at[j3], acc_sh.at[cidx.at[i % 4]],
                         sems.at[i % 2], add=True)
        return 0
    lax.fori_loop(0, NB, blk, 0)

    for t in range(2):
        i = NB - 2 + t
        pltpu.make_async_copy(rbuf.at[i % 3], acc_sh.at[cidx.at[i % 4]],
                              sems.at[i % 2]).wait()
    plsc.subcore_barrier()

    @pl.when(s < NS - 1)
    def _():
        pltpu.sync_copy(acc_sh.at[pl.ds(s * 624, 624)],
                        out_h.at[c, pl.ds(s * 624, 624)])

    @pl.when(s == NS - 1)
    def _():
        pltpu.sync_copy(acc_sh.at[pl.ds(15 * 624, 640)],
                        out_h.at[c, pl.ds(15 * 624, 640)])


def _mm_body(x_ref, w_ref, o_ref):
    o_ref[...] = lax.dot_general(
        x_ref[...], w_ref[...], (((1,), (1,)), ((), ())),
        preferred_element_type=jnp.float32)


def _tc_matmul(x, W):
    return pl.pallas_call(
        _mm_body,
        grid=(10,),
        in_specs=[pl.BlockSpec((N // 10, D), lambda i: (i, 0)),
                  pl.BlockSpec((D, D), lambda i: (0, 0))],
        out_specs=pl.BlockSpec((N // 10, D), lambda i: (i, 0)),
        out_shape=jax.ShapeDtypeStruct((N, D), jnp.float32),
    )(x, W)


def _mm2_body(p_ref, b_ref, w_ref, o_ref):
    t = jnp.maximum(p_ref[0] + p_ref[1] + b_ref[...], 0.0)
    o_ref[...] = lax.dot_general(
        t, w_ref[...], (((1,), (1,)), ((), ())),
        preferred_element_type=jnp.float32)


def _tc_combine_matmul(p, b, W):
    return pl.pallas_call(
        _mm2_body,
        grid=(10,),
        in_specs=[pl.BlockSpec((NC, N // 10, D), lambda i: (0, i, 0)),
                  pl.BlockSpec((1, D), lambda i: (0, 0)),
                  pl.BlockSpec((D, D), lambda i: (0, 0))],
        out_specs=pl.BlockSpec((N // 10, D), lambda i: (i, 0)),
        out_shape=jax.ShapeDtypeStruct((N, D), jnp.float32),
    )(p, b, W)


def _fin_body(p_ref, b_ref, o_ref):
    o_ref[...] = jnp.maximum(p_ref[0] + p_ref[1] + b_ref[...], 0.0)


def _tc_combine_relu(p, b):
    return pl.pallas_call(
        _fin_body,
        grid=(10,),
        in_specs=[pl.BlockSpec((NC, N // 10, D), lambda i: (0, i, 0)),
                  pl.BlockSpec((1, D), lambda i: (0, 0))],
        out_specs=pl.BlockSpec((N // 10, D), lambda i: (i, 0)),
        out_shape=jax.ShapeDtypeStruct((N, D), jnp.float32),
    )(p, b)


def kernel(x, edge_index, edge_weights, W1, b1, W2, b2):
    row = edge_index[0]
    col = edge_index[1]
    loop = jnp.arange(N, dtype=row.dtype)
    npad = EPAD - ETOT
    pad_idx = jnp.arange(npad, dtype=row.dtype) % N
    rows_flat = jnp.concatenate([row, loop, pad_idx])
    cols_flat = jnp.concatenate([col, loop, pad_idx])
    ew_flat = jnp.concatenate([edge_weights, jnp.ones((N,), jnp.float32),
                               jnp.zeros((npad,), jnp.float32)])
    rows3 = rows_flat.reshape(NW, NB, B)
    cols16 = cols_flat.reshape(NS, NB16, B)
    ew16 = ew_flat.reshape(NS, NB16, B)

    norm_flat = _norm_kernel(rows3, cols16, ew16).reshape(EPAD)

    h1 = _tc_matmul(x, W1)
    p1 = _mp_kernel(h1, rows_flat, cols_flat, norm_flat)
    h2 = _tc_combine_matmul(p1, b1.reshape(1, D), W2)
    p2 = _mp_kernel(h2, rows_flat, cols_flat, norm_flat)
    return _tc_combine_relu(p2, b2.reshape(1, D))
